# baseline (speedup 1.0000x reference)
import functools

import jax
import jax.numpy as jnp
from jax import lax
from jax.experimental import pallas as pl
from jax.experimental.pallas import tpu as pltpu
from jax.experimental.pallas import tpu_sc as plsc

EMBED_DIM = 64
IDX_MINOR = 128
CHUNK_ROWS = 1024
GATHERS_PER_CHUNK = CHUNK_ROWS // IDX_MINOR


@functools.partial(jax.jit, static_argnames=("n_rows",))
def _sc_gather(weight, idx2d, n_rows):
    info = plsc.get_sparse_core_info()
    nc, ns = info.num_cores, info.num_subcores
    nw = nc * ns
    rows_per_w = n_rows // nw
    n_chunks = rows_per_w // CHUNK_ROWS
    idx_rows_per_chunk = CHUNK_ROWS // IDX_MINOR

    mesh = plsc.VectorSubcoreMesh(core_axis_name="c", subcore_axis_name="s")

    @functools.partial(
        pl.kernel,
        mesh=mesh,
        out_type=jax.ShapeDtypeStruct((n_rows, EMBED_DIM), jnp.float32),
        scratch_types=[
            pltpu.VMEM((idx_rows_per_chunk, IDX_MINOR), jnp.int32),
            pltpu.VMEM((CHUNK_ROWS, EMBED_DIM), jnp.float32),
            pltpu.SemaphoreType.DMA,
        ],
        compiler_params=pltpu.CompilerParams(use_tc_tiling_on_sc=False),
    )
    def k(table_hbm, idx_hbm, out_hbm, idx_v, rows_v, sem):
        wid = lax.axis_index("s") * nc + lax.axis_index("c")
        row_base = wid * rows_per_w
        idx_row_base = row_base // IDX_MINOR

        def body(i, carry):
            row_off = pl.multiple_of(row_base + i * CHUNK_ROWS, 8)
            idx_row_off = pl.multiple_of(
                idx_row_base + i * idx_rows_per_chunk, 8)
            pltpu.sync_copy(idx_hbm.at[pl.ds(idx_row_off, idx_rows_per_chunk)],
                            idx_v)
            handles = []
            for j in range(GATHERS_PER_CHUNK):
                handles.append(
                    pltpu.async_copy(
                        table_hbm.at[idx_v.at[j]],
                        rows_v.at[pl.ds(j * IDX_MINOR, IDX_MINOR)],
                        sem,
                    ))
            for h in handles:
                h.wait()
            pltpu.sync_copy(rows_v, out_hbm.at[pl.ds(row_off, CHUNK_ROWS)])
            return carry

        lax.fori_loop(0, n_chunks, body, 0)

    return k(weight, idx2d)


def kernel(input_, weight):
    b, s = input_.shape
    n_rows = b * s
    idx2d = input_.reshape(n_rows // IDX_MINOR, IDX_MINOR).astype(jnp.int32)
    out = _sc_gather(weight, idx2d, n_rows)
    return out.reshape(b, s, EMBED_DIM)


# speedup vs baseline: 1.0129x; 1.0129x over previous
import functools

import jax
import jax.numpy as jnp
from jax import lax
from jax.experimental import pallas as pl
from jax.experimental.pallas import tpu as pltpu
from jax.experimental.pallas import tpu_sc as plsc

EMBED_DIM = 64
IDX_MINOR = 128
CHUNK_ROWS = 512
GATHERS_PER_CHUNK = CHUNK_ROWS // IDX_MINOR
NBUF = 2


@functools.partial(jax.jit, static_argnames=("n_rows",))
def _sc_gather(weight, idx_flat, n_rows):
    info = plsc.get_sparse_core_info()
    nc, ns = info.num_cores, info.num_subcores
    nw = nc * ns
    rows_per_w = n_rows // nw
    n_steps = rows_per_w // CHUNK_ROWS
    n_outer = n_steps // NBUF

    mesh = plsc.VectorSubcoreMesh(core_axis_name="c", subcore_axis_name="s")

    @functools.partial(
        pl.kernel,
        mesh=mesh,
        out_type=jax.ShapeDtypeStruct((n_rows, EMBED_DIM), jnp.float32),
        scratch_types=[
            pltpu.VMEM((NBUF, CHUNK_ROWS), jnp.int32),
            pltpu.VMEM((NBUF, CHUNK_ROWS, EMBED_DIM), jnp.float32),
            pltpu.SemaphoreType.DMA,
            pltpu.SemaphoreType.DMA,
            pltpu.SemaphoreType.DMA,
            pltpu.SemaphoreType.DMA,
            pltpu.SemaphoreType.DMA,
        ],
        compiler_params=pltpu.CompilerParams(use_tc_tiling_on_sc=False),
    )
    def k(table_hbm, idx_hbm, out_hbm, idx_v, rows_v, sem_i0, sem_i1,
          sem_g, sem_s0, sem_s1):
        wid = lax.axis_index("s") * nc + lax.axis_index("c")
        base = wid * rows_per_w
        sem_i = (sem_i0, sem_i1)
        sem_s = (sem_s0, sem_s1)

        def idx_copy(step, buf):
            off = pl.multiple_of(base + step * CHUNK_ROWS, 8)
            return pltpu.make_async_copy(
                idx_hbm.at[pl.ds(off, CHUNK_ROWS)], idx_v.at[buf],
                sem_i[buf])

        def store_copy(step, buf):
            off = pl.multiple_of(base + step * CHUNK_ROWS, 8)
            return pltpu.make_async_copy(
                rows_v.at[buf], out_hbm.at[pl.ds(off, CHUNK_ROWS)],
                sem_s[buf])

        idx_copy(0, 0).start()
        idx_copy(1, 1).start()

        def outer(g, carry):
            for b in range(NBUF):
                step = g * NBUF + b
                idx_copy(step, b).wait()

                @pl.when(g > 0)
                def _():
                    store_copy(step - NBUF, b).wait()

                handles = []
                for j in range(GATHERS_PER_CHUNK):
                    handles.append(
                        pltpu.async_copy(
                            table_hbm.at[idx_v.at[b, pl.ds(j * IDX_MINOR,
                                                           IDX_MINOR)]],
                            rows_v.at[b, pl.ds(j * IDX_MINOR, IDX_MINOR)],
                            sem_g,
                        ))
                for h in handles:
                    h.wait()

                store_copy(step, b).start()

                @pl.when(g < n_outer - 1)
                def _():
                    idx_copy(step + NBUF, b).start()
            return carry

        lax.fori_loop(0, n_outer, outer, 0)

        for b in range(NBUF):
            store_copy(n_steps - NBUF + b, b).wait()

    return k(weight, idx_flat)


def kernel(input_, weight):
    b, s = input_.shape
    n_rows = b * s
    idx_flat = input_.reshape(n_rows).astype(jnp.int32)
    out = _sc_gather(weight, idx_flat, n_rows)
    return out.reshape(b, s, EMBED_DIM)
